# baseline (speedup 1.0000x reference)
import functools

import jax
import jax.numpy as jnp
from jax import lax
from jax.experimental import pallas as pl
from jax.experimental.pallas import tpu as pltpu
from jax.experimental.pallas import tpu_sc as plsc

CHUNK = 64
LANES = 16
PACK = 8


def _hw_body(h_ref, w_ref, b_ref, o_ref):
    o_ref[...] = (
        jnp.dot(h_ref[...], w_ref[...], preferred_element_type=jnp.float32)
        + b_ref[...]
    )


def _xew_body(xe_ref, w_ref, o_ref):
    o_ref[...] = jnp.dot(xe_ref[...], w_ref[...], preferred_element_type=jnp.float32)


def _upd_body(h_ref, z0_ref, z1_ref, wh_ref, wz_ref, b_ref, o_ref):
    acc = jnp.dot(h_ref[...], wh_ref[...], preferred_element_type=jnp.float32)
    acc = acc + jnp.dot(
        z0_ref[...] + z1_ref[...], wz_ref[...], preferred_element_type=jnp.float32
    )
    o_ref[...] = jnp.maximum(acc + b_ref[...], 0.0)


@functools.cache
def _make_sc_edge_kernel(e_pad: int, nz: int, d: int):
    info = plsc.get_sparse_core_info()
    nc, ns = info.num_cores, info.num_subcores
    nw = nc * ns
    n_chunks = e_pad // CHUNK
    chunks_per_w = n_chunks // nw
    n_pairs = chunks_per_w // 2
    rows_per_tile = nz // ns
    d_slices = d // LANES
    rows_pk = CHUNK // PACK
    dpk = PACK * d
    mesh = plsc.VectorSubcoreMesh(core_axis_name="c", subcore_axis_name="s")

    @functools.partial(
        pl.kernel,
        out_type=jax.ShapeDtypeStruct((nc, nz, d), jnp.float32),
        mesh=mesh,
        scratch_types=[
            pltpu.VMEM((CHUNK,), jnp.int32),
            pltpu.VMEM((CHUNK,), jnp.int32),
            pltpu.VMEM((CHUNK,), jnp.int32),
            pltpu.VMEM((CHUNK,), jnp.int32),
            pltpu.VMEM((CHUNK, d), jnp.float32),
            pltpu.VMEM((CHUNK, d), jnp.float32),
            pltpu.VMEM((rows_pk, dpk), jnp.float32),
            pltpu.VMEM((rows_pk, dpk), jnp.float32),
            pltpu.VMEM((CHUNK, d), jnp.float32),
            pltpu.VMEM((CHUNK, d), jnp.float32),
            pltpu.VMEM_SHARED((nz, d), jnp.float32),
            pltpu.SemaphoreType.DMA,
            pltpu.SemaphoreType.DMA,
            pltpu.SemaphoreType.DMA,
            pltpu.SemaphoreType.DMA,
            pltpu.SemaphoreType.DMA,
            pltpu.SemaphoreType.DMA,
            pltpu.SemaphoreType.DMA,
            pltpu.SemaphoreType.DMA,
            pltpu.SemaphoreType.DMA,
            pltpu.SemaphoreType.DMA,
        ],
    )
    def sc_edge_kernel(hw_hbm, xew_hbm, src_hbm, dst_hbm, zpart_hbm,
                       srcb0, srcb1, dstb0, dstb1, rows0, rows1, xb0, xb1,
                       yb0, yb1, z_sh,
                       si0, si1, di0, di1, sg0, sg1, sx0, sx1, ss0, ss1):
        cid = lax.axis_index("c")
        sid = lax.axis_index("s")
        wid = sid * nc + cid
        c0 = wid * chunks_per_w
        srcb = (srcb0, srcb1)
        dstb = (dstb0, dstb1)
        rows = (rows0, rows1)
        xb = (xb0, xb1)
        yb = (yb0, yb1)
        si = (si0, si1)
        di = (di0, di1)
        sg = (sg0, sg1)
        sx = (sx0, sx1)
        ss = (ss0, ss1)

        def _issue_src(c, b):
            pltpu.async_copy(src_hbm.at[pl.ds((c0 + c) * CHUNK, CHUNK)], srcb[b], si[b])

        def _wait_src(c, b):
            pltpu.make_async_copy(
                src_hbm.at[pl.ds((c0 + c) * CHUNK, CHUNK)], srcb[b], si[b]
            ).wait()

        def _issue_dst(c, b):
            pltpu.async_copy(dst_hbm.at[pl.ds((c0 + c) * CHUNK, CHUNK)], dstb[b], di[b])

        def _wait_dst(c, b):
            pltpu.make_async_copy(
                dst_hbm.at[pl.ds((c0 + c) * CHUNK, CHUNK)], dstb[b], di[b]
            ).wait()

        def _issue_gather(b):
            pltpu.async_copy(hw_hbm.at[srcb[b]], rows[b], sg[b])

        def _wait_gather(b):
            pltpu.make_async_copy(hw_hbm.at[srcb[b]], rows[b], sg[b]).wait()

        def _issue_xew(c, b):
            pltpu.async_copy(
                xew_hbm.at[pl.ds((c0 + c) * rows_pk, rows_pk)], xb[b], sx[b]
            )

        def _wait_xew(c, b):
            pltpu.make_async_copy(
                xew_hbm.at[pl.ds((c0 + c) * rows_pk, rows_pk)], xb[b], sx[b]
            ).wait()

        def _compute(b):
            rows_b, x_b, y_b = rows[b], xb[b], yb[b]

            def _edge(j):
                jp = j // PACK
                jc = (j % PACK) * d
                for k in range(d_slices):
                    sl = pl.ds(k * LANES, LANES)
                    xsl = pl.ds(jc + k * LANES, LANES)
                    y_b[j, sl] = jnp.maximum(rows_b[j, sl] + x_b[jp, xsl], 0.0)

            plsc.parallel_loop(0, CHUNK, unroll=2)(_edge)

        def _issue_scatter(b):
            pltpu.async_copy(yb[b], z_sh.at[dstb[b]], ss[b], add=True)

        def _wait_scatter(b):
            pltpu.make_async_copy(yb[b], z_sh.at[dstb[b]], ss[b]).wait()

        zvec = jnp.zeros((LANES,), jnp.float32)

        def _zero_y(j, _):
            for k in range(d_slices):
                yb0[j, pl.ds(k * LANES, LANES)] = zvec
                yb1[j, pl.ds(k * LANES, LANES)] = zvec
            return 0

        lax.fori_loop(0, CHUNK, _zero_y, 0)

        z_base = sid * rows_per_tile
        n_full = rows_per_tile // CHUNK
        for r in range(n_full):
            pltpu.sync_copy(yb0, z_sh.at[pl.ds(z_base + r * CHUNK, CHUNK)])
        rem = rows_per_tile - n_full * CHUNK
        if rem:
            pltpu.sync_copy(
                yb0.at[pl.ds(0, rem)],
                z_sh.at[pl.ds(z_base + n_full * CHUNK, rem)],
            )
        plsc.subcore_barrier()

        last = chunks_per_w - 1
        for b in (0, 1):
            _issue_src(b, b)
            _issue_dst(b, b)
        for b in (0, 1):
            _wait_src(b, b)
            _issue_gather(b)
            _issue_xew(b, b)
        _wait_gather(0)
        _wait_xew(0, 0)
        _issue_src(2, 0)
        _compute(0)
        _wait_dst(0, 0)
        _issue_scatter(0)
        _wait_src(2, 0)
        _issue_gather(0)
        _issue_xew(2, 0)
        _wait_gather(1)
        _wait_xew(1, 1)
        _issue_src(3, 1)
        _wait_scatter(0)
        _issue_dst(2, 0)
        _compute(1)
        _wait_dst(1, 1)
        _issue_scatter(1)
        _wait_src(3, 1)
        _issue_gather(1)
        _issue_xew(3, 1)

        def _step(c, cn1, cn2, b):
            _wait_gather(b)
            _wait_xew(c, b)
            _issue_src(cn2, b)
            _wait_scatter(1 - b)
            _issue_dst(cn1, 1 - b)
            _compute(b)
            _wait_dst(c, b)
            _issue_scatter(b)
            _wait_src(cn2, b)
            _issue_gather(b)
            _issue_xew(cn2, b)

        def _pair(p, _):
            ca = 2 * p
            _step(ca, jnp.minimum(ca + 1, last), jnp.minimum(ca + 2, last), 0)
            _step(ca + 1, jnp.minimum(ca + 2, last), jnp.minimum(ca + 3, last), 1)
            return 0

        lax.fori_loop(1, n_pairs, _pair, 0)
        _wait_scatter(1)
        for b in (0, 1):
            _wait_gather(b)
            _wait_xew(last, b)
        _wait_dst(last, 0)
        plsc.subcore_barrier()

        pltpu.sync_copy(
            z_sh.at[pl.ds(z_base, rows_per_tile)],
            zpart_hbm.at[cid, pl.ds(z_base, rows_per_tile)],
        )

    return sc_edge_kernel


def _round_up(x: int, m: int) -> int:
    return (x + m - 1) // m * m


def kernel(H, Xe, id_Xe, W_M, b_M, W_U, b_U):
    n, d = H.shape
    e, de = Xe.shape
    info = plsc.get_sparse_core_info()
    nw = info.num_cores * info.num_subcores

    e_pad = _round_up(e, 2 * CHUNK * nw)
    nz = _round_up(n + 1, d)

    src = id_Xe[0].astype(jnp.int32)
    dst = id_Xe[1].astype(jnp.int32)
    n_pad = e_pad - e
    if n_pad:
        src = jnp.concatenate([src, jnp.zeros((n_pad,), jnp.int32)])
        pad_dst = n + jnp.arange(n_pad, dtype=jnp.int32) % (nz - n)
        dst = jnp.concatenate([dst, pad_dst])
        Xe = jnp.concatenate([Xe, jnp.zeros((n_pad, de), Xe.dtype)])

    w_mh, w_me = W_M[:d], W_M[d:]
    w_uh, w_uz = W_U[:d], W_U[d:]
    b_m2 = b_M.reshape(1, d)
    b_u2 = b_U.reshape(1, d)

    hw = pl.pallas_call(
        _hw_body,
        out_shape=jax.ShapeDtypeStruct((n, d), jnp.float32),
    )(H, w_mh, b_m2)

    xe_pk = Xe.reshape(e_pad // PACK, PACK * de)
    w_big = jnp.kron(jnp.eye(PACK, dtype=jnp.float32), w_me)
    n_pk = e_pad // PACK
    xew_blk = next(blk for blk in (4096, 2528, 2048, 1024, 512, 8)
                   if n_pk % blk == 0)
    xew = pl.pallas_call(
        _xew_body,
        grid=(n_pk // xew_blk,),
        in_specs=[
            pl.BlockSpec((xew_blk, PACK * de), lambda i: (i, 0)),
            pl.BlockSpec((PACK * de, PACK * d), lambda i: (0, 0)),
        ],
        out_specs=pl.BlockSpec((xew_blk, PACK * d), lambda i: (i, 0)),
        out_shape=jax.ShapeDtypeStruct((n_pk, PACK * d), jnp.float32),
    )(xe_pk, w_big)

    zpart = _make_sc_edge_kernel(e_pad, nz, d)(hw, xew, src, dst)

    z0 = lax.slice(zpart, (0, 0, 0), (1, n, d)).reshape(n, d)
    z1 = lax.slice(zpart, (1, 0, 0), (2, n, d)).reshape(n, d)

    out = pl.pallas_call(
        _upd_body,
        out_shape=jax.ShapeDtypeStruct((n, d), jnp.float32),
    )(H, z0, z1, w_uh, w_uz, b_u2)
    return out


# speedup vs baseline: 1.4507x; 1.4507x over previous
import functools

import jax
import jax.numpy as jnp
from jax import lax
from jax.experimental import pallas as pl
from jax.experimental.pallas import tpu as pltpu
from jax.experimental.pallas import tpu_sc as plsc

CHUNK = 40
LANES = 16


def _hw_body(h_ref, w_ref, b_ref, o_ref):
    o_ref[...] = (
        jnp.dot(h_ref[...], w_ref[...], preferred_element_type=jnp.float32)
        + b_ref[...]
    )


def _xew_body(xe_ref, w_ref, o_ref):
    o_ref[...] = jnp.dot(xe_ref[...], w_ref[...], preferred_element_type=jnp.float32)


def _upd_body(h_ref, z0_ref, z1_ref, wh_ref, wz_ref, b_ref, o_ref):
    acc = jnp.dot(h_ref[...], wh_ref[...], preferred_element_type=jnp.float32)
    acc = acc + jnp.dot(
        z0_ref[...] + z1_ref[...], wz_ref[...], preferred_element_type=jnp.float32
    )
    o_ref[...] = jnp.maximum(acc + b_ref[...], 0.0)


@functools.cache
def _make_sc_edge_kernel(e_pad: int, nz: int, d: int):
    info = plsc.get_sparse_core_info()
    nc, ns = info.num_cores, info.num_subcores
    nw = nc * ns
    n_chunks = e_pad // CHUNK
    chunks_per_w = n_chunks // nw
    n_pairs = chunks_per_w // 2
    rows_per_tile = nz // ns
    d_slices = d // LANES
    mesh = plsc.VectorSubcoreMesh(core_axis_name="c", subcore_axis_name="s")

    @functools.partial(
        pl.kernel,
        out_type=jax.ShapeDtypeStruct((nc, nz, d), jnp.float32),
        mesh=mesh,
        scratch_types=[
            pltpu.VMEM((CHUNK,), jnp.int32),
            pltpu.VMEM((CHUNK,), jnp.int32),
            pltpu.VMEM((CHUNK,), jnp.int32),
            pltpu.VMEM((CHUNK,), jnp.int32),
            pltpu.VMEM((CHUNK, d), jnp.float32),
            pltpu.VMEM((CHUNK, d), jnp.float32),
            pltpu.VMEM((CHUNK, d), jnp.float32),
            pltpu.VMEM((CHUNK, d), jnp.float32),
            pltpu.VMEM((CHUNK, d), jnp.float32),
            pltpu.VMEM((CHUNK, d), jnp.float32),
            pltpu.VMEM_SHARED((nz, d), jnp.float32),
            pltpu.SemaphoreType.DMA,
            pltpu.SemaphoreType.DMA,
            pltpu.SemaphoreType.DMA,
            pltpu.SemaphoreType.DMA,
            pltpu.SemaphoreType.DMA,
            pltpu.SemaphoreType.DMA,
            pltpu.SemaphoreType.DMA,
            pltpu.SemaphoreType.DMA,
            pltpu.SemaphoreType.DMA,
            pltpu.SemaphoreType.DMA,
        ],
    )
    def sc_edge_kernel(hw_hbm, xew_hbm, src_hbm, dst_hbm, zpart_hbm,
                       srcb0, srcb1, dstb0, dstb1, rows0, rows1, xb0, xb1,
                       yb0, yb1, z_sh,
                       si0, si1, di0, di1, sg0, sg1, sx0, sx1, ss0, ss1):
        cid = lax.axis_index("c")
        sid = lax.axis_index("s")
        wid = sid * nc + cid
        c0 = wid * chunks_per_w
        srcb = (srcb0, srcb1)
        dstb = (dstb0, dstb1)
        rows = (rows0, rows1)
        xb = (xb0, xb1)
        yb = (yb0, yb1)
        si = (si0, si1)
        di = (di0, di1)
        sg = (sg0, sg1)
        sx = (sx0, sx1)
        ss = (ss0, ss1)

        def _issue_src(c, b):
            pltpu.async_copy(src_hbm.at[pl.ds((c0 + c) * CHUNK, CHUNK)], srcb[b], si[b])

        def _wait_src(c, b):
            pltpu.make_async_copy(
                src_hbm.at[pl.ds((c0 + c) * CHUNK, CHUNK)], srcb[b], si[b]
            ).wait()

        def _issue_dst(c, b):
            pltpu.async_copy(dst_hbm.at[pl.ds((c0 + c) * CHUNK, CHUNK)], dstb[b], di[b])

        def _wait_dst(c, b):
            pltpu.make_async_copy(
                dst_hbm.at[pl.ds((c0 + c) * CHUNK, CHUNK)], dstb[b], di[b]
            ).wait()

        def _issue_gather(b):
            pltpu.async_copy(hw_hbm.at[srcb[b]], rows[b], sg[b])

        def _wait_gather(b):
            pltpu.make_async_copy(hw_hbm.at[srcb[b]], rows[b], sg[b]).wait()

        def _issue_xew(c, b):
            pltpu.async_copy(xew_hbm.at[pl.ds((c0 + c) * CHUNK, CHUNK)], xb[b], sx[b])

        def _wait_xew(c, b):
            pltpu.make_async_copy(
                xew_hbm.at[pl.ds((c0 + c) * CHUNK, CHUNK)], xb[b], sx[b]
            ).wait()

        def _compute(b):
            rows_b, x_b, y_b = rows[b], xb[b], yb[b]

            def _edge(j):
                for k in range(d_slices):
                    sl = pl.ds(k * LANES, LANES)
                    y_b[j, sl] = jnp.maximum(rows_b[j, sl] + x_b[j, sl], 0.0)

            plsc.parallel_loop(0, CHUNK, unroll=2)(_edge)

        def _issue_scatter(b):
            pltpu.async_copy(yb[b], z_sh.at[dstb[b]], ss[b], add=True)

        def _wait_scatter(b):
            pltpu.make_async_copy(yb[b], z_sh.at[dstb[b]], ss[b]).wait()

        zvec = jnp.zeros((LANES,), jnp.float32)

        def _zero_y(j, _):
            for k in range(d_slices):
                yb0[j, pl.ds(k * LANES, LANES)] = zvec
                yb1[j, pl.ds(k * LANES, LANES)] = zvec
            return 0

        lax.fori_loop(0, CHUNK, _zero_y, 0)

        def _zero_z(r, _):
            pltpu.sync_copy(
                yb0, z_sh.at[pl.ds(sid * rows_per_tile + r * CHUNK, CHUNK)]
            )
            return 0

        lax.fori_loop(0, rows_per_tile // CHUNK, _zero_z, 0)
        plsc.subcore_barrier()

        last = chunks_per_w - 1
        for b in (0, 1):
            _issue_src(b, b)
            _issue_dst(b, b)
        for b in (0, 1):
            _wait_src(b, b)
            _issue_gather(b)
            _issue_xew(b, b)
        _wait_gather(0)
        _wait_xew(0, 0)
        _issue_src(2, 0)
        _compute(0)
        _wait_dst(0, 0)
        _issue_scatter(0)
        _wait_src(2, 0)
        _issue_gather(0)
        _issue_xew(2, 0)
        _wait_gather(1)
        _wait_xew(1, 1)
        _issue_src(3, 1)
        _wait_scatter(0)
        _issue_dst(2, 0)
        _compute(1)
        _wait_dst(1, 1)
        _issue_scatter(1)
        _wait_src(3, 1)
        _issue_gather(1)
        _issue_xew(3, 1)

        def _step(c, cn1, cn2, b):
            _wait_gather(b)
            _wait_xew(c, b)
            _issue_src(cn2, b)
            _wait_scatter(1 - b)
            _issue_dst(cn1, 1 - b)
            _compute(b)
            _wait_dst(c, b)
            _issue_scatter(b)
            _wait_src(cn2, b)
            _issue_gather(b)
            _issue_xew(cn2, b)

        def _pair(p, _):
            ca = 2 * p
            _step(ca, jnp.minimum(ca + 1, last), jnp.minimum(ca + 2, last), 0)
            _step(ca + 1, jnp.minimum(ca + 2, last), jnp.minimum(ca + 3, last), 1)
            return 0

        lax.fori_loop(1, n_pairs, _pair, 0)
        _wait_scatter(1)
        for b in (0, 1):
            _wait_gather(b)
            _wait_xew(last, b)
        _wait_dst(last, 0)
        plsc.subcore_barrier()

        pltpu.sync_copy(
            z_sh.at[pl.ds(sid * rows_per_tile, rows_per_tile)],
            zpart_hbm.at[cid, pl.ds(sid * rows_per_tile, rows_per_tile)],
        )

    return sc_edge_kernel


def _round_up(x: int, m: int) -> int:
    return (x + m - 1) // m * m


def kernel(H, Xe, id_Xe, W_M, b_M, W_U, b_U):
    n, d = H.shape
    e, de = Xe.shape
    info = plsc.get_sparse_core_info()
    nw = info.num_cores * info.num_subcores

    e_pad = _round_up(e, 2 * CHUNK * nw)
    nz = _round_up(n + 1, info.num_subcores * CHUNK)

    src = id_Xe[0].astype(jnp.int32)
    dst = id_Xe[1].astype(jnp.int32)
    n_pad = e_pad - e
    if n_pad:
        src = jnp.concatenate([src, jnp.zeros((n_pad,), jnp.int32)])
        pad_dst = n + jnp.arange(n_pad, dtype=jnp.int32) % (nz - n)
        dst = jnp.concatenate([dst, pad_dst])
        Xe = jnp.concatenate([Xe, jnp.zeros((n_pad, de), Xe.dtype)])

    w_mh, w_me = W_M[:d], W_M[d:]
    w_uh, w_uz = W_U[:d], W_U[d:]
    b_m2 = b_M.reshape(1, d)
    b_u2 = b_U.reshape(1, d)

    hw = pl.pallas_call(
        _hw_body,
        out_shape=jax.ShapeDtypeStruct((n, d), jnp.float32),
    )(H, w_mh, b_m2)

    xew_blk = next(blk for blk in (4000, 4096, 2048, 1280, 2 * CHUNK)
                   if e_pad % blk == 0)
    xew = pl.pallas_call(
        _xew_body,
        grid=(e_pad // xew_blk,),
        in_specs=[
            pl.BlockSpec((xew_blk, de), lambda i: (i, 0)),
            pl.BlockSpec((de, d), lambda i: (0, 0)),
        ],
        out_specs=pl.BlockSpec((xew_blk, d), lambda i: (i, 0)),
        out_shape=jax.ShapeDtypeStruct((e_pad, d), jnp.float32),
    )(Xe, w_me)

    zpart = _make_sc_edge_kernel(e_pad, nz, d)(hw, xew, src, dst)

    z0 = lax.slice(zpart, (0, 0, 0), (1, n, d)).reshape(n, d)
    z1 = lax.slice(zpart, (1, 0, 0), (2, n, d)).reshape(n, d)

    out = pl.pallas_call(
        _upd_body,
        out_shape=jax.ShapeDtypeStruct((n, d), jnp.float32),
    )(H, z0, z1, w_uh, w_uz, b_u2)
    return out
